# baseline (speedup 1.0000x reference)
import functools

import jax
import jax.numpy as jnp
from jax import lax
from jax.experimental import pallas as pl
from jax.experimental.pallas import tpu as pltpu
from jax.experimental.pallas import tpu_sc as plsc

_RB = 64
_NC = 2
_NS = 16
_LANES = 16


def _argmax_kernel(x_ref, idx_ref):
    xb = x_ref[...]
    bmax = jnp.max(xb, axis=1, keepdims=True)
    iota = lax.broadcasted_iota(jnp.int32, xb.shape, 1)
    idx_ref[...] = jnp.min(
        jnp.where(xb == bmax, iota, xb.shape[1]), axis=1, keepdims=True
    )


def _make_zerofill(rows, cols):
    rpw = rows // (_NC * _NS)
    mesh = plsc.VectorSubcoreMesh(core_axis_name="c", subcore_axis_name="s")

    @functools.partial(
        pl.kernel,
        out_type=jax.ShapeDtypeStruct((rows, cols), jnp.float32),
        mesh=mesh,
        scratch_types=[pltpu.VMEM((1, cols), jnp.float32)],
    )
    def zerofill(out_hbm, zrow_v):
        wid = lax.axis_index("s") * _NC + lax.axis_index("c")
        z16 = jnp.zeros((_LANES,), jnp.float32)

        @pl.loop(0, cols // _LANES)
        def _zero_row(i):
            zrow_v[0, pl.ds(i * _LANES, _LANES)] = z16

        base = wid * rpw
        for r in range(rpw):
            pltpu.sync_copy(zrow_v, out_hbm.at[pl.ds(base + r, 1)])

    return zerofill


def _scatter_kernel(z_hbm, idx_s, idxv_ref, out_hbm, e_v, patch_v, sem):
    del z_hbm
    rows = idxv_ref.shape[0]
    idxv = idxv_ref[...]
    iota = lax.broadcasted_iota(jnp.int32, (rows, 128), 1)
    e_v[...] = (iota == (idxv & 127)).astype(jnp.float32)
    e = e_v[...]
    mv = idxv >> 7
    copies = []
    for i in range(rows):
        k = i // 8
        m_i = idx_s[i, 0] >> 7
        rowmask = (mv[k * 8:(k + 1) * 8] == m_i).astype(jnp.float32)
        patch_v[8 * i:8 * i + 8, :] = e[k * 8:(k + 1) * 8, :] * rowmask
        col0 = pl.multiple_of(m_i * 128, 128)
        cp = pltpu.make_async_copy(
            patch_v.at[pl.ds(8 * i, 8)],
            out_hbm.at[pl.ds(8 * k, 8), pl.ds(col0, 128)],
            sem,
        )
        cp.start()
        copies.append(cp)
    for cp in copies:
        cp.wait()


def kernel(x):
    rows, cols = x.shape
    idx = pl.pallas_call(
        _argmax_kernel,
        grid=(rows // _RB,),
        in_specs=[pl.BlockSpec((_RB, cols), lambda i: (i, 0))],
        out_specs=pl.BlockSpec((_RB, 1), lambda i: (i, 0)),
        out_shape=jax.ShapeDtypeStruct((rows, 1), jnp.int32),
    )(x)
    zeros = _make_zerofill(rows, cols)()
    return pl.pallas_call(
        _scatter_kernel,
        in_specs=[
            pl.BlockSpec(memory_space=pl.ANY),
            pl.BlockSpec(memory_space=pltpu.SMEM),
            pl.BlockSpec((rows, 1), lambda: (0, 0)),
        ],
        out_specs=pl.BlockSpec(memory_space=pl.ANY),
        out_shape=jax.ShapeDtypeStruct((rows, cols), jnp.float32),
        input_output_aliases={0: 0},
        scratch_shapes=[
            pltpu.VMEM((rows, 128), jnp.float32),
            pltpu.VMEM((rows * 8, 128), jnp.float32),
            pltpu.SemaphoreType.DMA,
        ],
    )(zeros, idx, idx)


# speedup vs baseline: 3.2753x; 3.2753x over previous
import jax
import jax.numpy as jnp
from jax.experimental import pallas as pl

_RB = 64


def _ste_kernel(x_ref, out_ref):
    xb = x_ref[...]
    bmax = jnp.max(xb, axis=1, keepdims=True)
    iota = jax.lax.broadcasted_iota(jnp.int32, xb.shape, 1)
    bidx = jnp.min(
        jnp.where(xb == bmax, iota, xb.shape[1]), axis=1, keepdims=True
    )
    out_ref[...] = (iota == bidx).astype(jnp.float32)


def kernel(x):
    rows, cols = x.shape
    return pl.pallas_call(
        _ste_kernel,
        grid=(rows // _RB,),
        in_specs=[pl.BlockSpec((_RB, cols), lambda i: (i, 0))],
        out_specs=pl.BlockSpec((_RB, cols), lambda i: (i, 0)),
        out_shape=jax.ShapeDtypeStruct((rows, cols), jnp.float32),
    )(x)
